# baseline (speedup 1.0000x reference)
import jax
import jax.numpy as jnp
from jax import lax
from jax.experimental import pallas as pl
from jax.experimental.pallas import tpu as pltpu
from jax.experimental.pallas import tpu_sc as plsc

NC = 2
NS = 16
LANES = 16
CHUNK = 128
IB = 32


def _sc_mesh():
    return plsc.VectorSubcoreMesh(core_axis_name="c", subcore_axis_name="s")


def _worker(NS_):
    c = lax.axis_index("c")
    s = lax.axis_index("s")
    return c, s, c * NS_ + s


def _zero_shared(sh, zsrc, s, NPAD, width_ref_rows):
    CZ = -(-NPAD // CHUNK)
    for k in range(-(-CZ // NS)):
        off = jnp.minimum((k * NS + s) * CHUNK, NPAD - CHUNK)
        pltpu.sync_copy(zsrc, sh.at[pl.ds(off, CHUNK)])


def _writeout(sh, out, c, s, N, NS_):
    rpt8 = (N // NS_) // 8 * 8
    wrem = N - rpt8 * NS_
    obase = pl.multiple_of(s * rpt8, 8)
    pltpu.sync_copy(sh.at[pl.ds(obase, rpt8)], out.at[c, pl.ds(obase, rpt8)])
    if wrem:
        @pl.when(s == NS_ - 1)
        def _tail():
            tb = NS_ * rpt8
            pltpu.sync_copy(sh.at[pl.ds(tb, wrem)], out.at[c, pl.ds(tb, wrem)])


def _make_sc_agg(N, D, RT):
    NPAD = N + LANES
    DH = D // 2
    assert N % 8 == 0 and DH % LANES == 0 and RT % IB == 0

    out_type = jax.ShapeDtypeStruct((NC, N, DH), jnp.float32)
    scratch = [
        pltpu.VMEM_SHARED((N, DH), jnp.float32),
        pltpu.VMEM_SHARED((NPAD, DH), jnp.float32),
        pltpu.VMEM((IB, CHUNK), jnp.int32),
        pltpu.VMEM((IB, CHUNK), jnp.int32),
        pltpu.VMEM((CHUNK, DH), jnp.float32),
        pltpu.VMEM((CHUNK, DH), jnp.float32),
        pltpu.SemaphoreType.DMA,
        pltpu.SemaphoreType.DMA,
    ]

    def body(x_hbm, src_hbm, dst_hbm, agg_out,
             xsh, agg_sh, srcv, dstv, rows0, rows1, sem0, sem1):
        c, s, w = _worker(NS)
        rows = (rows0, rows1)
        sems = (sem0, sem1)

        z16 = jnp.zeros((LANES,), jnp.float32)

        def zrow_body(i, carry):
            for j in range(DH // LANES):
                rows0[i, pl.ds(j * LANES, LANES)] = z16
            return carry

        lax.fori_loop(0, CHUNK, zrow_body, 0)
        _zero_shared(agg_sh, rows0, s, NPAD, DH)

        rpt8 = (N // NS) // 8 * 8
        wrem = N - rpt8 * NS
        obase = pl.multiple_of(s * rpt8, 8)
        pltpu.sync_copy(x_hbm.at[c, pl.ds(obase, rpt8)],
                        xsh.at[pl.ds(obase, rpt8)])
        if wrem:
            @pl.when(s == NS - 1)
            def _tail_stage():
                tb = NS * rpt8
                pltpu.sync_copy(x_hbm.at[c, pl.ds(tb, wrem)],
                                xsh.at[pl.ds(tb, wrem)])
        plsc.subcore_barrier()

        def block_body(b, carry):
            boff = pl.multiple_of(s * RT + b * IB, IB)
            pltpu.sync_copy(src_hbm.at[pl.ds(boff, IB)], srcv)
            pltpu.sync_copy(dst_hbm.at[pl.ds(boff, IB)], dstv)
            pend = pltpu.async_copy(xsh.at[srcv.at[0]], rows[0], sems[0])
            for jj in range(IB):
                pend.wait()
                if jj + 1 < IB:
                    pend = pltpu.async_copy(
                        xsh.at[srcv.at[jj + 1]],
                        rows[(jj + 1) % 2], sems[(jj + 1) % 2])
                pltpu.sync_copy(rows[jj % 2], agg_sh.at[dstv.at[jj]],
                                add=True)
            return carry

        lax.fori_loop(0, RT // IB, block_body, 0)
        plsc.subcore_barrier()
        _writeout(agg_sh, agg_out, c, s, N, NS)

    return pl.kernel(body, out_type=out_type, mesh=_sc_mesh(),
                     scratch_types=scratch)


def _make_sc_count(N, DC, RC):
    NPAD = N + LANES
    NB = RC // IB
    assert RC % IB == 0

    out_type = jax.ShapeDtypeStruct((NC, N, DC), jnp.float32)
    scratch = [
        pltpu.VMEM_SHARED((NPAD, DC), jnp.float32),
        pltpu.VMEM((IB, CHUNK), jnp.int32),
        pltpu.VMEM((CHUNK, DC), jnp.float32),
    ]

    def body(dst_hbm, cnt_out, cnt_sh, dstv, ones):
        c, s, w = _worker(NS)

        def fill_body(val):
            v16 = jnp.full((LANES,), val, jnp.float32)

            def fb(i, carry):
                for j in range(DC // LANES):
                    ones[i, pl.ds(j * LANES, LANES)] = v16
                return carry

            return fb

        lax.fori_loop(0, CHUNK, fill_body(0.0), 0)
        _zero_shared(cnt_sh, ones, s, NPAD, DC)
        lax.fori_loop(0, CHUNK, fill_body(1.0), 0)
        plsc.subcore_barrier()

        def block_body(b, carry):
            boff = pl.multiple_of(w * RC + b * IB, IB)
            pltpu.sync_copy(dst_hbm.at[pl.ds(boff, IB)], dstv)
            for jj in range(IB):
                pltpu.sync_copy(ones, cnt_sh.at[dstv.at[jj]], add=True)
            return carry

        lax.fori_loop(0, NB, block_body, 0)
        plsc.subcore_barrier()
        _writeout(cnt_sh, cnt_out, c, s, N, NS)

    return pl.kernel(body, out_type=out_type, mesh=_sc_mesh(),
                     scratch_types=scratch)


def _make_tc_layer(N, D, B, relu, out_stacked):
    assert N % B == 0
    DH = D // 2

    def body(agg_ref, cnt_ref, x_ref, wlt_ref, bl_ref, wrt_ref, o_ref):
        a = jnp.concatenate([agg_ref[0], agg_ref[1]], axis=1)
        xf = jnp.concatenate([x_ref[0], x_ref[1]], axis=1)
        cn = cnt_ref[0, :, 0:1] + cnt_ref[1, :, 0:1]
        mean = a / jnp.maximum(cn, 1.0)
        t = jnp.dot(mean, wlt_ref[...], preferred_element_type=jnp.float32)
        t = t + bl_ref[...] + jnp.dot(xf, wrt_ref[...],
                                      preferred_element_type=jnp.float32)
        t = jnp.maximum(t, 0.0) if relu else t
        if out_stacked:
            o_ref[0] = t[:, :DH]
            o_ref[1] = t[:, DH:]
        else:
            o_ref[...] = t

    if out_stacked:
        out_spec = pl.BlockSpec((NC, B, DH), lambda i: (0, i, 0))
        out_shape = jax.ShapeDtypeStruct((NC, N, DH), jnp.float32)
    else:
        out_spec = pl.BlockSpec((B, D), lambda i: (i, 0))
        out_shape = jax.ShapeDtypeStruct((N, D), jnp.float32)

    return pl.pallas_call(
        body,
        grid=(N // B,),
        in_specs=[
            pl.BlockSpec((NC, B, DH), lambda i: (0, i, 0)),
            pl.BlockSpec((NC, B, DH), lambda i: (0, i, 0)),
            pl.BlockSpec((NC, B, DH), lambda i: (0, i, 0)),
            pl.BlockSpec((D, D), lambda i: (0, 0)),
            pl.BlockSpec((1, D), lambda i: (0, 0)),
            pl.BlockSpec((D, D), lambda i: (0, 0)),
        ],
        out_specs=out_spec,
        out_shape=out_shape,
    )


def kernel(x, edge_index, W1l, b1l, W1r, W2l, b2l, W2r):
    N, D = x.shape
    E = edge_index.shape[1]
    NW = NC * NS
    R = -(-E // (NW * CHUNK))
    R = -(-R // IB) * IB
    TCH = NW * R
    EP = TCH * CHUNK
    pad = EP - E

    RT = TCH // NS

    src = edge_index[0]
    dst = edge_index[1]
    if pad:
        src = jnp.concatenate([src, jnp.zeros((pad,), jnp.int32)])
        dst = jnp.concatenate([dst, jnp.full((pad,), N, jnp.int32)])
    src2 = src.reshape(TCH, CHUNK)
    dst2 = dst.reshape(TCH, CHUNK)

    B = 2000
    DH = D // 2
    xs = jnp.stack([x[:, :DH], x[:, DH:]])
    cnt = _make_sc_count(N, DH, R)(dst2)
    agg1 = _make_sc_agg(N, D, RT)(xs, src2, dst2)
    hs = _make_tc_layer(N, D, B, True, True)(
        agg1, cnt, xs, W1l.T, b1l.reshape(1, D), W1r.T)
    agg2 = _make_sc_agg(N, D, RT)(hs, src2, dst2)
    out = _make_tc_layer(N, D, B, False, False)(
        agg2, cnt, hs, W2l.T, b2l.reshape(1, D), W2r.T)
    return out


# speedup vs baseline: 1.3186x; 1.3186x over previous
import jax
import jax.numpy as jnp
from jax import lax
from jax.experimental import pallas as pl
from jax.experimental.pallas import tpu as pltpu
from jax.experimental.pallas import tpu_sc as plsc

NC = 2
NS = 16
LANES = 16
CHUNK = 128
IB = 16


def _sc_mesh():
    return plsc.VectorSubcoreMesh(core_axis_name="c", subcore_axis_name="s")


def _worker(NS_):
    c = lax.axis_index("c")
    s = lax.axis_index("s")
    return c, s, c * NS_ + s


def _zero_shared(sh, zsrc, s, NPAD, width_ref_rows):
    CZ = -(-NPAD // CHUNK)
    for k in range(-(-CZ // NS)):
        off = jnp.minimum((k * NS + s) * CHUNK, NPAD - CHUNK)
        pltpu.sync_copy(zsrc, sh.at[pl.ds(off, CHUNK)])


def _writeout(sh, out, c, s, N, NS_):
    rpt8 = (N // NS_) // 8 * 8
    wrem = N - rpt8 * NS_
    obase = pl.multiple_of(s * rpt8, 8)
    pltpu.sync_copy(sh.at[pl.ds(obase, rpt8)], out.at[c, pl.ds(obase, rpt8)])
    if wrem:
        @pl.when(s == NS_ - 1)
        def _tail():
            tb = NS_ * rpt8
            pltpu.sync_copy(sh.at[pl.ds(tb, wrem)], out.at[c, pl.ds(tb, wrem)])


def _make_sc_agg(N, D, RT):
    NPAD = N + LANES
    DH = D // 2
    assert N % 8 == 0 and DH % LANES == 0 and RT % IB == 0

    out_type = jax.ShapeDtypeStruct((NC, N, DH), jnp.float32)
    scratch = [
        pltpu.VMEM_SHARED((N, DH), jnp.float32),
        pltpu.VMEM_SHARED((NPAD, DH), jnp.float32),
        pltpu.VMEM((IB, CHUNK), jnp.int32),
        pltpu.VMEM((IB, CHUNK), jnp.int32),
        pltpu.VMEM((CHUNK, DH), jnp.float32),
        pltpu.VMEM((CHUNK, DH), jnp.float32),
        pltpu.SemaphoreType.DMA,
        pltpu.SemaphoreType.DMA,
    ]

    def body(x_hbm, src_hbm, dst_hbm, agg_out,
             xsh, agg_sh, srcv, dstv, rows0, rows1, sem0, sem1):
        c, s, w = _worker(NS)
        rows = (rows0, rows1)
        sems = (sem0, sem1)

        z16 = jnp.zeros((LANES,), jnp.float32)

        def zrow_body(i, carry):
            for j in range(DH // LANES):
                rows0[i, pl.ds(j * LANES, LANES)] = z16
            return carry

        lax.fori_loop(0, CHUNK, zrow_body, 0)
        _zero_shared(agg_sh, rows0, s, NPAD, DH)

        rpt8 = (N // NS) // 8 * 8
        wrem = N - rpt8 * NS
        obase = pl.multiple_of(s * rpt8, 8)
        pltpu.sync_copy(x_hbm.at[c, pl.ds(obase, rpt8)],
                        xsh.at[pl.ds(obase, rpt8)])
        if wrem:
            @pl.when(s == NS - 1)
            def _tail_stage():
                tb = NS * rpt8
                pltpu.sync_copy(x_hbm.at[c, pl.ds(tb, wrem)],
                                xsh.at[pl.ds(tb, wrem)])
        plsc.subcore_barrier()

        def block_body(b, carry):
            boff = pl.multiple_of(s * RT + b * IB, IB)
            pltpu.sync_copy(src_hbm.at[pl.ds(boff, IB)], srcv)
            pltpu.sync_copy(dst_hbm.at[pl.ds(boff, IB)], dstv)
            pend = pltpu.async_copy(xsh.at[srcv.at[0]], rows[0], sems[0])
            for jj in range(IB):
                pend.wait()
                if jj + 1 < IB:
                    pend = pltpu.async_copy(
                        xsh.at[srcv.at[jj + 1]],
                        rows[(jj + 1) % 2], sems[(jj + 1) % 2])
                pltpu.sync_copy(rows[jj % 2], agg_sh.at[dstv.at[jj]],
                                add=True)
            return carry

        lax.fori_loop(0, RT // IB, block_body, 0)
        plsc.subcore_barrier()
        _writeout(agg_sh, agg_out, c, s, N, NS)

    return pl.kernel(body, out_type=out_type, mesh=_sc_mesh(),
                     scratch_types=scratch)


def _make_sc_count(N, DC, RC):
    NPAD = N + LANES
    NB = RC // IB
    assert RC % IB == 0

    out_type = jax.ShapeDtypeStruct((NC, N, DC), jnp.float32)
    scratch = [
        pltpu.VMEM_SHARED((NPAD, DC), jnp.float32),
        pltpu.VMEM((IB, CHUNK), jnp.int32),
        pltpu.VMEM((CHUNK, DC), jnp.float32),
    ]

    def body(dst_hbm, cnt_out, cnt_sh, dstv, ones):
        c, s, w = _worker(NS)

        def fill_body(val):
            v16 = jnp.full((LANES,), val, jnp.float32)

            def fb(i, carry):
                for j in range(DC // LANES):
                    ones[i, pl.ds(j * LANES, LANES)] = v16
                return carry

            return fb

        lax.fori_loop(0, CHUNK, fill_body(0.0), 0)
        _zero_shared(cnt_sh, ones, s, NPAD, DC)
        lax.fori_loop(0, CHUNK, fill_body(1.0), 0)
        plsc.subcore_barrier()

        def block_body(b, carry):
            boff = pl.multiple_of(w * RC + b * IB, IB)
            pltpu.sync_copy(dst_hbm.at[pl.ds(boff, IB)], dstv)
            for jj in range(IB):
                pltpu.sync_copy(ones, cnt_sh.at[dstv.at[jj]], add=True)
            return carry

        lax.fori_loop(0, NB, block_body, 0)
        plsc.subcore_barrier()
        _writeout(cnt_sh, cnt_out, c, s, N, NS)

    return pl.kernel(body, out_type=out_type, mesh=_sc_mesh(),
                     scratch_types=scratch)


def _make_tc_layer(N, D, DC, B, relu, out_stacked):
    assert N % B == 0
    DH = D // 2

    def body(agg_ref, cnt_ref, x_ref, wlt_ref, bl_ref, wrt_ref, o_ref):
        a = jnp.concatenate([agg_ref[0], agg_ref[1]], axis=1)
        xf = jnp.concatenate([x_ref[0], x_ref[1]], axis=1)
        cn = cnt_ref[0, :, 0:1] + cnt_ref[1, :, 0:1]
        mean = a / jnp.maximum(cn, 1.0)
        t = jnp.dot(mean, wlt_ref[...], preferred_element_type=jnp.float32)
        t = t + bl_ref[...] + jnp.dot(xf, wrt_ref[...],
                                      preferred_element_type=jnp.float32)
        t = jnp.maximum(t, 0.0) if relu else t
        if out_stacked:
            o_ref[0] = t[:, :DH]
            o_ref[1] = t[:, DH:]
        else:
            o_ref[...] = t

    if out_stacked:
        out_spec = pl.BlockSpec((NC, B, DH), lambda i: (0, i, 0))
        out_shape = jax.ShapeDtypeStruct((NC, N, DH), jnp.float32)
    else:
        out_spec = pl.BlockSpec((B, D), lambda i: (i, 0))
        out_shape = jax.ShapeDtypeStruct((N, D), jnp.float32)

    return pl.pallas_call(
        body,
        grid=(N // B,),
        in_specs=[
            pl.BlockSpec((NC, B, DH), lambda i: (0, i, 0)),
            pl.BlockSpec((NC, B, DC), lambda i: (0, i, 0)),
            pl.BlockSpec((NC, B, DH), lambda i: (0, i, 0)),
            pl.BlockSpec((D, D), lambda i: (0, 0)),
            pl.BlockSpec((1, D), lambda i: (0, 0)),
            pl.BlockSpec((D, D), lambda i: (0, 0)),
        ],
        out_specs=out_spec,
        out_shape=out_shape,
    )


def kernel(x, edge_index, W1l, b1l, W1r, W2l, b2l, W2r):
    N, D = x.shape
    E = edge_index.shape[1]
    NW = NC * NS
    R = -(-E // (NW * CHUNK))
    R = -(-R // IB) * IB
    TCH = NW * R
    EP = TCH * CHUNK
    pad = EP - E

    RT = TCH // NS

    src = edge_index[0]
    dst = edge_index[1]
    if pad:
        src = jnp.concatenate([src, jnp.zeros((pad,), jnp.int32)])
        dst = jnp.concatenate([dst, jnp.full((pad,), N, jnp.int32)])
    src2 = src.reshape(TCH, CHUNK)
    dst2 = dst.reshape(TCH, CHUNK)

    B = 2000
    DH = D // 2
    xs = jnp.stack([x[:, :DH], x[:, DH:]])
    DC = 32
    cnt = _make_sc_count(N, DC, R)(dst2)
    agg1 = _make_sc_agg(N, D, RT)(xs, src2, dst2)
    hs = _make_tc_layer(N, D, DC, B, True, True)(
        agg1, cnt, xs, W1l.T, b1l.reshape(1, D), W1r.T)
    agg2 = _make_sc_agg(N, D, RT)(hs, src2, dst2)
    out = _make_tc_layer(N, D, DC, B, False, False)(
        agg2, cnt, hs, W2l.T, b2l.reshape(1, D), W2r.T)
    return out
